# baseline (speedup 1.0000x reference)
import functools
import jax
import jax.numpy as jnp
from jax import lax
from jax.experimental import pallas as pl
from jax.experimental.pallas import tpu as pltpu
from jax.experimental.pallas import tpu_sc as plsc

D = 64
SCALE = 8.0

NC, NS = 2, 16
NW = NC * NS
BATCH = 4096
SEQ = 200
BCOL = BATCH // NW


def _emb_body(xt_hbm, lut2_hbm, out_hbm, idxv, idx2v, parv, gbuf, obuf, gsem, wsem):
    wid = lax.axis_index("s") * NC + lax.axis_index("c")
    bcol = wid * BCOL
    pltpu.sync_copy(xt_hbm.at[:, pl.ds(bcol, BCOL)], idxv)

    iota = jax.lax.iota(jnp.int32, 16)
    rowv = [iota + (16 * k) for k in range(8)]

    @pl.loop(0, SEQ)
    def _s(s):
        for k in range(8):
            sl = pl.ds(16 * k, 16)
            v = idxv[s, sl]
            idx2v[0, sl] = jax.lax.shift_right_logical(v, 1)
            parv[0, sl] = (v & 1) * 64
        pltpu.async_copy(lut2_hbm.at[idx2v.at[0]], gbuf, gsem).wait()
        for k in range(8):
            sl = pl.ds(16 * k, 16)
            colbase = parv[0, sl]

            @pl.loop(0, D)
            def _f(f):
                vals = plsc.load_gather(gbuf, [rowv[k], colbase + f])
                obuf[f, sl] = vals * SCALE

        pltpu.sync_copy(obuf, out_hbm.at[s, :, pl.ds(bcol, BCOL)])


@jax.jit
def _emb(xt, lut2):
    mesh = plsc.VectorSubcoreMesh(
        core_axis_name="c", subcore_axis_name="s", num_cores=NC, num_subcores=NS
    )
    run = pl.kernel(
        _emb_body,
        out_type=jax.ShapeDtypeStruct((SEQ, D, BATCH), jnp.float32),
        mesh=mesh,
        scratch_types=[
            pltpu.VMEM((SEQ, BCOL), jnp.int32),
            pltpu.VMEM((2, BCOL), jnp.int32),
            pltpu.VMEM((2, BCOL), jnp.int32),
            pltpu.VMEM((BCOL, 128), jnp.float32),
            pltpu.VMEM((D, BCOL), jnp.float32),
            pltpu.SemaphoreType.DMA,
            pltpu.SemaphoreType.DMA,
        ],
        compiler_params=pltpu.CompilerParams(
            use_tc_tiling_on_sc=True, needs_layout_passes=False
        ),
    )
    return run(xt, lut2)


def kernel(x, lut):
    xt = x.T.astype(jnp.int32)
    lut2 = lut.reshape(500000, 128)
    out_t = _emb(xt, lut2)
    return out_t.transpose(2, 0, 1)


# speedup vs baseline: 1.0791x; 1.0791x over previous
import functools
import jax
import jax.numpy as jnp
from jax import lax
from jax.experimental import pallas as pl
from jax.experimental.pallas import tpu as pltpu
from jax.experimental.pallas import tpu_sc as plsc

D = 64
SCALE = 8.0

NC, NS = 2, 16
NW = NC * NS
BATCH = 4096
SEQ = 200
BCOL = BATCH // NW


def _emb_body(xt_hbm, lut_hbm, out_hbm, idxv, gbuf0, gbuf1, obuf0, obuf1,
              gsem0, gsem1, wsem0, wsem1):
    wid = lax.axis_index("s") * NC + lax.axis_index("c")
    bcol = wid * BCOL
    gbufs, obufs = (gbuf0, gbuf1), (obuf0, obuf1)
    gsems, wsems = (gsem0, gsem1), (wsem0, wsem1)

    pltpu.sync_copy(xt_hbm.at[:, pl.ds(bcol, BCOL)], idxv)

    iota = jax.lax.iota(jnp.int32, 16)
    rowv = [iota + (16 * k) for k in range(8)]

    def fire_gather(s, b):
        pltpu.async_copy(lut_hbm.at[idxv.at[s]], gbufs[b], gsems[b])

    fire_gather(0, 0)

    @pl.loop(0, SEQ, step=2)
    def _s(g):
        for b in range(2):
            s = g + b
            gbuf, obuf = gbufs[b], obufs[b]

            @pl.when(s + 1 < SEQ)
            def _():
                fire_gather(s + 1, 1 - b)

            pltpu.make_async_copy(lut_hbm.at[pl.ds(0, BCOL)], gbuf, gsems[b]).wait()

            @pl.when(s >= 2)
            def _():
                pltpu.make_async_copy(
                    obuf, out_hbm.at[s, :, pl.ds(bcol, BCOL)], wsems[b]
                ).wait()

            @pl.loop(0, D)
            def _f(f):
                fv = jnp.full((16,), f, jnp.int32)
                for k in range(8):
                    vals = plsc.load_gather(gbuf, [rowv[k], fv])
                    obuf[f, pl.ds(16 * k, 16)] = vals * SCALE

            pltpu.async_copy(obuf, out_hbm.at[s, :, pl.ds(bcol, BCOL)], wsems[b])

    for b in range(2):
        pltpu.make_async_copy(
            obufs[b], out_hbm.at[0, :, pl.ds(bcol, BCOL)], wsems[b]
        ).wait()


@jax.jit
def _emb(xt, lut):
    mesh = plsc.VectorSubcoreMesh(
        core_axis_name="c", subcore_axis_name="s", num_cores=NC, num_subcores=NS
    )
    run = pl.kernel(
        _emb_body,
        out_type=jax.ShapeDtypeStruct((SEQ, D, BATCH), jnp.float32),
        mesh=mesh,
        scratch_types=[
            pltpu.VMEM((SEQ, BCOL), jnp.int32),
            pltpu.VMEM((BCOL, D), jnp.float32),
            pltpu.VMEM((BCOL, D), jnp.float32),
            pltpu.VMEM((D, BCOL), jnp.float32),
            pltpu.VMEM((D, BCOL), jnp.float32),
            pltpu.SemaphoreType.DMA,
            pltpu.SemaphoreType.DMA,
            pltpu.SemaphoreType.DMA,
            pltpu.SemaphoreType.DMA,
        ],
        compiler_params=pltpu.CompilerParams(
            use_tc_tiling_on_sc=False, needs_layout_passes=False
        ),
    )
    return run(xt, lut)


def kernel(x, lut):
    xt = x.T.astype(jnp.int32)
    out_t = _emb(xt, lut)
    return out_t.transpose(2, 0, 1)


# speedup vs baseline: 1.7937x; 1.6622x over previous
import functools
import jax
import jax.numpy as jnp
from jax import lax
from jax.experimental import pallas as pl
from jax.experimental.pallas import tpu as pltpu
from jax.experimental.pallas import tpu_sc as plsc

D = 64
SCALE = 8.0

NC, NS = 2, 16
NW = NC * NS
BATCH = 4096
SEQ = 200
BCOL = BATCH // NW


def _emb_body(xt_hbm, lut_hbm, out_hbm, idxv, gbuf0, gbuf1, obuf0, obuf1,
              gsem0, gsem1, wsem0, wsem1):
    wid = lax.axis_index("s") * NC + lax.axis_index("c")
    bcol = wid * BCOL
    gbufs, obufs = (gbuf0, gbuf1), (obuf0, obuf1)
    gsems, wsems = (gsem0, gsem1), (wsem0, wsem1)

    pltpu.sync_copy(xt_hbm.at[:, pl.ds(bcol, BCOL)], idxv)

    iota = jax.lax.iota(jnp.int32, 16)
    rowv = [iota + (16 * k) for k in range(8)]

    def fire_gather(s, b):
        pltpu.async_copy(lut_hbm.at[idxv.at[s]], gbufs[b], gsems[b])

    fire_gather(0, 0)

    @pl.loop(0, SEQ, step=2)
    def _s(g):
        for b in range(2):
            s = g + b
            gbuf, obuf = gbufs[b], obufs[b]

            @pl.when(s + 1 < SEQ)
            def _():
                fire_gather(s + 1, 1 - b)

            pltpu.make_async_copy(lut_hbm.at[pl.ds(0, BCOL)], gbuf, gsems[b]).wait()

            @pl.when(s >= 2)
            def _():
                pltpu.make_async_copy(
                    obuf, out_hbm.at[s, :, pl.ds(bcol, BCOL)], wsems[b]
                ).wait()

            @pl.loop(0, D)
            def _f(f):
                for k in range(8):
                    vals = gbuf[f, pl.ds(16 * (k % 4), 16)]
                    obuf[f, pl.ds(16 * k, 16)] = vals * SCALE

            pltpu.async_copy(obuf, out_hbm.at[s, :, pl.ds(bcol, BCOL)], wsems[b])

    for b in range(2):
        pltpu.make_async_copy(
            obufs[b], out_hbm.at[0, :, pl.ds(bcol, BCOL)], wsems[b]
        ).wait()


@jax.jit
def _emb(xt, lut):
    mesh = plsc.VectorSubcoreMesh(
        core_axis_name="c", subcore_axis_name="s", num_cores=NC, num_subcores=NS
    )
    run = pl.kernel(
        _emb_body,
        out_type=jax.ShapeDtypeStruct((SEQ, D, BATCH), jnp.float32),
        mesh=mesh,
        scratch_types=[
            pltpu.VMEM((SEQ, BCOL), jnp.int32),
            pltpu.VMEM((BCOL, D), jnp.float32),
            pltpu.VMEM((BCOL, D), jnp.float32),
            pltpu.VMEM((D, BCOL), jnp.float32),
            pltpu.VMEM((D, BCOL), jnp.float32),
            pltpu.SemaphoreType.DMA,
            pltpu.SemaphoreType.DMA,
            pltpu.SemaphoreType.DMA,
            pltpu.SemaphoreType.DMA,
        ],
        compiler_params=pltpu.CompilerParams(
            use_tc_tiling_on_sc=False, needs_layout_passes=False
        ),
    )
    return run(xt, lut)


def kernel(x, lut):
    xt = x.T.astype(jnp.int32)
    out_t = _emb(xt, lut)
    return out_t.transpose(2, 0, 1)


# speedup vs baseline: 2.4786x; 1.3818x over previous
import functools
import jax
import jax.numpy as jnp
from jax import lax
from jax.experimental import pallas as pl
from jax.experimental.pallas import tpu as pltpu
from jax.experimental.pallas import tpu_sc as plsc

D_MODEL = 64
SCALE = 8.0

NC, NS = 2, 16
NW = NC * NS
RG = 128
G = 5
CH = G * RG

B = 4096 * 200
B_PER_W = B // NW
GROUPS_PER_W = B_PER_W // RG
CHUNKS_PER_W = B_PER_W // CH


def _emb_body(x_hbm, lut_hbm, out_hbm, idx_v, buf0, buf1, gsem0, gsem1):
    wid = lax.axis_index("s") * NC + lax.axis_index("c")
    gbase = wid * GROUPS_PER_W
    pltpu.sync_copy(x_hbm.at[pl.ds(gbase, GROUPS_PER_W)], idx_v)

    bufs = (buf0, buf1)
    sems = (gsem0, gsem1)

    def fire(chunk, buf, sem):
        for j in range(G):
            pltpu.async_copy(
                lut_hbm.at[idx_v.at[chunk * G + j]],
                buf.at[pl.ds(j * RG, RG)],
                sem,
            )

    def drain(buf, sem):
        pltpu.make_async_copy(lut_hbm.at[pl.ds(0, CH)], buf, sem).wait()

    fire(0, buf0, gsem0)

    @pl.loop(0, CHUNKS_PER_W, step=2)
    def _chunk(g):
        for b in range(2):
            chunk = g + b
            buf, sem = bufs[b], sems[b]

            @pl.when(chunk + 1 < CHUNKS_PER_W)
            def _():
                fire(chunk + 1, bufs[1 - b], sems[1 - b])

            drain(buf, sem)

            @pl.loop(0, CH, unroll=8)
            def _row(i):
                for j in range(D_MODEL // 16):
                    sl = pl.ds(j * 16, 16)
                    buf[i, sl] = buf[i, sl] * SCALE

            crow = (gbase + chunk * G) * RG
            pltpu.sync_copy(buf, out_hbm.at[pl.ds(crow, CH), pl.ds(0, D_MODEL)])


@jax.jit
def _emb(x2, lut):
    mesh = plsc.VectorSubcoreMesh(
        core_axis_name="c", subcore_axis_name="s", num_cores=NC, num_subcores=NS
    )
    run = pl.kernel(
        _emb_body,
        out_type=jax.ShapeDtypeStruct((B, 128), jnp.float32),
        mesh=mesh,
        scratch_types=[
            pltpu.VMEM((GROUPS_PER_W, RG), jnp.int32),
            pltpu.VMEM((CH, D_MODEL), jnp.float32),
            pltpu.VMEM((CH, D_MODEL), jnp.float32),
            pltpu.SemaphoreType.DMA,
            pltpu.SemaphoreType.DMA,
        ],
        compiler_params=pltpu.CompilerParams(
            use_tc_tiling_on_sc=False, needs_layout_passes=False
        ),
    )
    return run(x2, lut)


def kernel(x, lut):
    x2 = x.reshape(B // RG, RG).astype(jnp.int32)
    out = _emb(x2, lut)
    return out[:, :D_MODEL].reshape(x.shape[0], x.shape[1], D_MODEL)
